# Initial kernel scaffold; baseline (speedup 1.0000x reference)
import functools

import jax
import jax.numpy as jnp
from jax import lax
from jax.experimental import pallas as pl
from jax.experimental.pallas import tpu as pltpu
from jax.experimental.pallas import tpu_sc as plsc

P = 524288
D = 16
BUCKETS = 4194304
MASK = BUCKETS - 1
RES = 0.02
SMIN = -1.0

NC, NS = 2, 16
NW = NC * NS
PT = P // NW
C = 512
NG = C // 16
NI = PT // C

_P2 = 2654435761
_P3 = 805459861
_CORN = [(0, 0, 0), (1, 0, 0), (0, 1, 0), (1, 1, 0),
         (0, 0, 1), (1, 0, 1), (0, 1, 1), (1, 1, 1)]


def _corner_off(c):
    o = (c[0] + c[1] * _P2 + c[2] * _P3) & 0xFFFFFFFF
    return o - (1 << 32) if o >= (1 << 31) else o


_OFFS = [_corner_off(c) for c in _CORN]
_K2 = _P2 - (1 << 32)
_K3 = _P3


def _floor_i32(q):
    bi = q.astype(jnp.int32)
    bf = bi.astype(jnp.float32)
    bi = bi - (bf > q).astype(jnp.int32)
    bf = bi.astype(jnp.float32)
    return bi, bf


def _tln_body(pts_hbm, tab_hbm, out_hbm, pts_v, out_v, idxs, ws, rows, sem):
    wid = lax.axis_index("s") * NC + lax.axis_index("c")
    lane = lax.iota(jnp.int32, 16)
    tile_base = wid * PT

    def chunk(i, carry):
        base = tile_base + i * C
        pltpu.sync_copy(pts_hbm.at[pl.ds(base, C)], pts_v)

        def hash_group(g, c2):
            p0 = g * 16
            pidx = p0 + lane
            z16 = jnp.zeros((16,), jnp.int32)
            px = plsc.load_gather(pts_v, [pidx, z16])
            py = plsc.load_gather(pts_v, [pidx, z16 + 1])
            pz = plsc.load_gather(pts_v, [pidx, z16 + 2])
            qx = (px - jnp.float32(SMIN)) / jnp.float32(RES)
            qy = (py - jnp.float32(SMIN)) / jnp.float32(RES)
            qz = (pz - jnp.float32(SMIN)) / jnp.float32(RES)
            bix, bfx = _floor_i32(qx)
            biy, bfy = _floor_i32(qy)
            biz, bfz = _floor_i32(qz)
            t = bix + biy * jnp.int32(_K2) + biz * jnp.int32(_K3)
            fx = qx - bfx
            fy = qy - bfy
            fz = qz - bfz
            wxs = (jnp.float32(1.0) - fx, fx)
            wys = (jnp.float32(1.0) - fy, fy)
            wzs = (jnp.float32(1.0) - fz, fz)
            for c in range(8):
                cx, cy, cz = _CORN[c]
                vid = (t + jnp.int32(_OFFS[c])) & jnp.int32(MASK)
                idxs[c][pl.ds(p0, 16)] = vid
                ws[c][pl.ds(p0, 16)] = (wxs[cx] * wys[cy]) * wzs[cz]
            return c2

        lax.fori_loop(jnp.int32(0), jnp.int32(NG), hash_group, jnp.int32(0))

        cps = [pltpu.async_copy(tab_hbm.at[idxs[c]], rows[c], sem)
               for c in range(8)]
        for cp in cps:
            cp.wait()

        def interp_group(g, c2):
            p0 = g * 16
            pidx = p0 + lane
            wv = [ws[c][pl.ds(p0, 16)] for c in range(8)]
            for d in range(D):
                dd = jnp.full((16,), d, jnp.int32)
                acc = wv[0] * plsc.load_gather(rows[0], [pidx, dd])
                for c in range(1, 8):
                    acc = acc + wv[c] * plsc.load_gather(rows[c], [pidx, dd])
                plsc.store_scatter(out_v, [pidx, dd], acc)
            return c2

        lax.fori_loop(jnp.int32(0), jnp.int32(NG), interp_group, jnp.int32(0))

        pltpu.sync_copy(out_v, out_hbm.at[pl.ds(base, C)])
        return carry

    lax.fori_loop(jnp.int32(0), jnp.int32(NI), chunk, jnp.int32(0))


@jax.jit
def _run(pts, table):
    mesh = plsc.VectorSubcoreMesh(core_axis_name="c", subcore_axis_name="s")
    return pl.kernel(
        _tln_body,
        out_type=jax.ShapeDtypeStruct((P, D), jnp.float32),
        mesh=mesh,
        scratch_types=[
            pltpu.VMEM((C, 3), jnp.float32),
            pltpu.VMEM((C, D), jnp.float32),
            [pltpu.VMEM((C,), jnp.int32) for _ in range(8)],
            [pltpu.VMEM((C,), jnp.float32) for _ in range(8)],
            [pltpu.VMEM((C, D), jnp.float32) for _ in range(8)],
            pltpu.SemaphoreType.DMA,
        ],
    )(pts, table)


def kernel(pts, voxel_features):
    return _run(pts, voxel_features)


# speedup vs baseline: 1.5385x; 1.5385x over previous
import functools

import jax
import jax.numpy as jnp
from jax import lax
from jax.experimental import pallas as pl
from jax.experimental.pallas import tpu as pltpu
from jax.experimental.pallas import tpu_sc as plsc

P = 524288
D = 16
BUCKETS = 4194304
MASK = BUCKETS - 1
RES = 0.02
SMIN = -1.0

NC, NS = 2, 16
NW = NC * NS
PT = P // NW
C = 512
NG = C // 16
NI = PT // C

_P2 = 2654435761
_P3 = 805459861
_CORN = [(0, 0, 0), (1, 0, 0), (0, 1, 0), (1, 1, 0),
         (0, 0, 1), (1, 0, 1), (0, 1, 1), (1, 1, 1)]


def _corner_off(c):
    o = (c[0] + c[1] * _P2 + c[2] * _P3) & 0xFFFFFFFF
    return o - (1 << 32) if o >= (1 << 31) else o


_OFFS = [_corner_off(c) for c in _CORN]
_K2 = _P2 - (1 << 32)
_K3 = _P3


def _floor_i32(q):
    bi = q.astype(jnp.int32)
    bf = bi.astype(jnp.float32)
    bi = bi - (bf > q).astype(jnp.int32)
    bf = bi.astype(jnp.float32)
    return bi, bf


def _tln_body(pts_hbm, tab_hbm, out_hbm, pts_v, out_v, idxs, ws, rows, sem):
    wid = lax.axis_index("s") * NC + lax.axis_index("c")
    lane = lax.iota(jnp.int32, 16)
    tile_base = wid * PT

    def chunk(i, carry):
        base = tile_base + i * C
        pltpu.sync_copy(pts_hbm.at[pl.ds(base, C)], pts_v)

        def hash_group(g, c2):
            p0 = g * 16
            pidx = p0 + lane
            z16 = jnp.zeros((16,), jnp.int32)
            px = plsc.load_gather(pts_v, [pidx, z16])
            py = plsc.load_gather(pts_v, [pidx, z16 + 1])
            pz = plsc.load_gather(pts_v, [pidx, z16 + 2])
            qx = (px - jnp.float32(SMIN)) / jnp.float32(RES)
            qy = (py - jnp.float32(SMIN)) / jnp.float32(RES)
            qz = (pz - jnp.float32(SMIN)) / jnp.float32(RES)
            bix, bfx = _floor_i32(qx)
            biy, bfy = _floor_i32(qy)
            biz, bfz = _floor_i32(qz)
            t = bix + biy * jnp.int32(_K2) + biz * jnp.int32(_K3)
            fx = qx - bfx
            fy = qy - bfy
            fz = qz - bfz
            wxs = (jnp.float32(1.0) - fx, fx)
            wys = (jnp.float32(1.0) - fy, fy)
            wzs = (jnp.float32(1.0) - fz, fz)
            for c in range(8):
                cx, cy, cz = _CORN[c]
                vid = (t + jnp.int32(_OFFS[c])) & jnp.int32(MASK)
                idxs[c][pl.ds(p0, 16)] = vid
                ws[c][pl.ds(p0, 16)] = (wxs[cx] * wys[cy]) * wzs[cz]
            return c2

        lax.fori_loop(jnp.int32(0), jnp.int32(NG), hash_group, jnp.int32(0))

        cps = [pltpu.async_copy(tab_hbm.at[idxs[c]], rows[c], sem)
               for c in range(8)]
        for cp in cps:
            cp.wait()

        def interp_group(g, c2):
            p0 = g * 16
            pidx = p0 + lane
            wv = [ws[c][pl.ds(p0, 16)] for c in range(8)]
            for d in range(D):
                dd = jnp.full((16,), d, jnp.int32)
                acc = wv[0] * plsc.load_gather(rows[0], [pidx, dd])
                for c in range(1, 8):
                    acc = acc + wv[c] * plsc.load_gather(rows[c], [pidx, dd])
                plsc.store_scatter(out_v, [pidx, dd], acc)
            return c2

        lax.fori_loop(jnp.int32(0), jnp.int32(NG), interp_group, jnp.int32(0))

        pltpu.sync_copy(out_v, out_hbm.at[pl.ds(base, C)])
        return carry

    lax.fori_loop(jnp.int32(0), jnp.int32(NI), chunk, jnp.int32(0))


@jax.jit
def _run(pts, table):
    mesh = plsc.VectorSubcoreMesh(core_axis_name="c", subcore_axis_name="s")
    return pl.kernel(
        _tln_body,
        out_type=jax.ShapeDtypeStruct((P, D), jnp.float32),
        mesh=mesh,
        scratch_types=[
            pltpu.VMEM((C, 3), jnp.float32),
            pltpu.VMEM((C, D), jnp.float32),
            [pltpu.VMEM((C,), jnp.int32) for _ in range(8)],
            [pltpu.VMEM((C,), jnp.float32) for _ in range(8)],
            [pltpu.VMEM((C, D), jnp.float32) for _ in range(8)],
            pltpu.SemaphoreType.DMA,
        ],
        compiler_params=pltpu.CompilerParams(
            needs_layout_passes=False, use_tc_tiling_on_sc=False),
    )(pts, table)


def kernel(pts, voxel_features):
    return _run(pts, voxel_features)


# speedup vs baseline: 2.2884x; 1.4875x over previous
import functools

import jax
import jax.numpy as jnp
from jax import lax
from jax.experimental import pallas as pl
from jax.experimental.pallas import tpu as pltpu
from jax.experimental.pallas import tpu_sc as plsc

P = 524288
D = 16
BUCKETS = 4194304
MASK = BUCKETS - 1
RES = 0.02
SMIN = -1.0

NC, NS = 2, 16
NW = NC * NS
PT = P // NW
C = 256
NG = C // 16
NI = PT // C

_P2 = 2654435761
_P3 = 805459861
_CORN = [(0, 0, 0), (1, 0, 0), (0, 1, 0), (1, 1, 0),
         (0, 0, 1), (1, 0, 1), (0, 1, 1), (1, 1, 1)]


def _corner_off(c):
    o = (c[0] + c[1] * _P2 + c[2] * _P3) & 0xFFFFFFFF
    return o - (1 << 32) if o >= (1 << 31) else o


_OFFS = [_corner_off(c) for c in _CORN]
_K2 = _P2 - (1 << 32)
_K3 = _P3


def _floor_i32(q):
    bi = q.astype(jnp.int32)
    bf = bi.astype(jnp.float32)
    bi = bi - (bf > q).astype(jnp.int32)
    bf = bi.astype(jnp.float32)
    return bi, bf


def _tln_body(pts_hbm, tab_hbm, out_hbm, ptsb, idxb, wb, rowsb, out_v, sems):
    wid = lax.axis_index("s") * NC + lax.axis_index("c")
    lane = lax.iota(jnp.int32, 16)
    tile_base = wid * PT

    def hash_chunk(i, buf):
        pts_v, idxs, ws = ptsb[buf], idxb[buf], wb[buf]
        base = tile_base + i * C
        pltpu.sync_copy(pts_hbm.at[:, pl.ds(base, C)], pts_v)

        def group(g, c2):
            p0 = g * 16
            px = pts_v[0, pl.ds(p0, 16)]
            py = pts_v[1, pl.ds(p0, 16)]
            pz = pts_v[2, pl.ds(p0, 16)]
            qx = (px - jnp.float32(SMIN)) / jnp.float32(RES)
            qy = (py - jnp.float32(SMIN)) / jnp.float32(RES)
            qz = (pz - jnp.float32(SMIN)) / jnp.float32(RES)
            bix, bfx = _floor_i32(qx)
            biy, bfy = _floor_i32(qy)
            biz, bfz = _floor_i32(qz)
            t = bix + biy * jnp.int32(_K2) + biz * jnp.int32(_K3)
            fx = qx - bfx
            fy = qy - bfy
            fz = qz - bfz
            wxs = (jnp.float32(1.0) - fx, fx)
            wys = (jnp.float32(1.0) - fy, fy)
            wzs = (jnp.float32(1.0) - fz, fz)
            for c in range(8):
                cx, cy, cz = _CORN[c]
                vid = (t + jnp.int32(_OFFS[c])) & jnp.int32(MASK)
                idxs[c][pl.ds(p0, 16)] = vid
                ws[c][pl.ds(p0, 16)] = (wxs[cx] * wys[cy]) * wzs[cz]
            return c2

        lax.fori_loop(jnp.int32(0), jnp.int32(NG), group, jnp.int32(0))

    def issue(buf):
        for c in range(8):
            pltpu.async_copy(tab_hbm.at[idxb[buf][c]], rowsb[buf][c],
                             sems[buf])

    def drain(buf):
        for c in range(8):
            pltpu.make_async_copy(tab_hbm.at[pl.ds(0, C)], rowsb[buf][c],
                                  sems[buf]).wait()

    def interp_store(i, buf):
        rows, ws = rowsb[buf], wb[buf]
        base = tile_base + i * C

        def group(g, c2):
            p0 = g * 16
            pidx = p0 + lane
            wv = [ws[c][pl.ds(p0, 16)] for c in range(8)]
            for d in range(D):
                dd = jnp.full((16,), d, jnp.int32)
                acc = wv[0] * plsc.load_gather(rows[0], [pidx, dd])
                for c in range(1, 8):
                    acc = acc + wv[c] * plsc.load_gather(rows[c], [pidx, dd])
                out_v[d, pl.ds(p0, 16)] = acc
            return c2

        lax.fori_loop(jnp.int32(0), jnp.int32(NG), group, jnp.int32(0))
        pltpu.sync_copy(out_v, out_hbm.at[:, pl.ds(base, C)])

    hash_chunk(jnp.int32(0), 0)
    issue(0)

    def body(j, carry):
        i0 = j * 2
        hash_chunk(i0 + 1, 1)
        issue(1)
        drain(0)
        interp_store(i0, 0)

        @pl.when(j < jnp.int32(NI // 2 - 1))
        def _():
            hash_chunk(i0 + 2, 0)
            issue(0)

        drain(1)
        interp_store(i0 + 1, 1)
        return carry

    lax.fori_loop(jnp.int32(0), jnp.int32(NI // 2), body, jnp.int32(0))


@jax.jit
def _run(pts_t, table):
    mesh = plsc.VectorSubcoreMesh(core_axis_name="c", subcore_axis_name="s")
    out_t = pl.kernel(
        _tln_body,
        out_type=jax.ShapeDtypeStruct((D, P), jnp.float32),
        mesh=mesh,
        scratch_types=[
            [pltpu.VMEM((3, C), jnp.float32) for _ in range(2)],
            [[pltpu.VMEM((C,), jnp.int32) for _ in range(8)]
             for _ in range(2)],
            [[pltpu.VMEM((C,), jnp.float32) for _ in range(8)]
             for _ in range(2)],
            [[pltpu.VMEM((C, D), jnp.float32) for _ in range(8)]
             for _ in range(2)],
            pltpu.VMEM((D, C), jnp.float32),
            [pltpu.SemaphoreType.DMA for _ in range(2)],
        ],
        compiler_params=pltpu.CompilerParams(
            needs_layout_passes=False, use_tc_tiling_on_sc=False),
    )(pts_t, table)
    return out_t.T


def kernel(pts, voxel_features):
    return _run(pts.T, voxel_features)
